# baseline (speedup 1.0000x reference)
import functools

import jax
import jax.numpy as jnp
from jax.experimental import pallas as pl
from jax.experimental.pallas import tpu as pltpu

N = 10000
L = 2
K = (L + 1) ** 2
KP = K + 1
MAXNN = 32
RADIUS = 0.1
MAXITER = 2

NPAD = 10240
PB = 512
RNG = 1024
NROWS = 12288


def _sh2(u):
    x = u[..., 0]; y = u[..., 1]; z = u[..., 2]
    return jnp.stack([
        0.282095 * jnp.ones_like(x),
        0.488603 * y,
        0.488603 * z,
        0.488603 * x,
        1.092548 * x * y,
        1.092548 * y * z,
        0.315392 * (3.0 * z * z - 1.0),
        1.092548 * x * z,
        0.546274 * (x * x - y * y),
    ], axis=-1)


def _build_graph(xyz, radius, max_nn):
    n = xyz.shape[0]
    xn = jnp.sum(xyz * xyz, -1)
    idxs = []
    valids = []
    chunk = 2500
    for s in range(0, n, chunk):
        q = xyz[s:s + chunk]
        qn = jnp.sum(q * q, -1)
        d2 = qn[:, None] + xn[None, :] - 2.0 * (q @ xyz.T)
        d2 = jnp.maximum(d2, 0.0)
        rows = s + jnp.arange(q.shape[0])
        d2 = jnp.where(jnp.arange(n)[None, :] == rows[:, None], jnp.inf, d2)
        vals, idx = jax.lax.top_k(-d2, max_nn)
        valid = (-vals) < radius * radius
        idxs.append(idx)
        valids.append(valid)
    nn_idx = jnp.concatenate(idxs, 0)
    valid = jnp.concatenate(valids, 0)
    nn_cnt = valid.sum(-1)
    dirs = xyz[nn_idx] - xyz[:, None, :]
    r = jnp.sqrt(jnp.maximum(jnp.sum(dirs * dirs, -1), 1e-12))
    u = dirs / r[..., None]
    sh = _sh2(u)
    coeff = jnp.concatenate([sh, jnp.ones(sh.shape[:-1] + (1,), dtype=sh.dtype)], axis=-1)
    coeff = coeff * valid[..., None].astype(coeff.dtype)
    return nn_cnt, nn_idx, coeff


def _scatter_body(v0s_ref, sv_ref, fcs_ref, f1g_ref, acc_ref):
    b = pl.program_id(0)

    @pl.when(b == 0)
    def _():
        acc_ref[...] = jnp.zeros_like(acc_ref)

    c1 = f1g_ref.shape[1]
    v0 = pl.multiple_of(v0s_ref[b], 8)
    rel = sv_ref[0, :, :] - v0
    rel = jnp.clip(rel, 0, RNG - 1)
    onehot_t = (jax.lax.broadcasted_iota(jnp.int32, (RNG, PB), 0)
                == rel).astype(jnp.float32)
    f1g = f1g_ref[...]
    z = jnp.concatenate(
        [fcs_ref[:, k][:, None] * f1g for k in range(K)], axis=1)
    delta = jnp.dot(onehot_t, z, preferred_element_type=jnp.float32)
    cur = acc_ref[pl.ds(v0, RNG), :]
    acc_ref[pl.ds(v0, RNG), :] = cur + delta


@functools.partial(jax.jit, static_argnames=("c1",))
def _scatter_acc(v0s, sv3, fcs, f1g, c1):
    nb = sv3.shape[0] // PB
    grid_spec = pltpu.PrefetchScalarGridSpec(
        num_scalar_prefetch=1,
        grid=(nb,),
        in_specs=[
            pl.BlockSpec((1, 1, PB), lambda b, v0s: (b, 0, 0)),
            pl.BlockSpec((PB, 16), lambda b, v0s: (b, 0)),
            pl.BlockSpec((PB, c1), lambda b, v0s: (b, 0)),
        ],
        out_specs=pl.BlockSpec((NROWS, K * c1), lambda b, v0s: (0, 0)),
    )
    return pl.pallas_call(
        _scatter_body,
        grid_spec=grid_spec,
        out_shape=jax.ShapeDtypeStruct((NROWS, K * c1), jnp.float32),
        compiler_params=pltpu.CompilerParams(
            dimension_semantics=("arbitrary",)),
    )(v0s, sv3.reshape(nb, 1, PB), fcs, f1g)


def _prep_pairs(face, filt_coeff):
    f = face.shape[0]
    npairs = f * 3
    npad = ((npairs + PB - 1) // PB) * PB
    pair_v = face.reshape(-1)
    order = jnp.argsort(pair_v)
    sv = jnp.concatenate(
        [pair_v[order], jnp.full((npad - npairs,), NPAD, jnp.int32)])
    pair_f = jnp.concatenate(
        [(order // 3).astype(jnp.int32), jnp.zeros((npad - npairs,), jnp.int32)])
    fc_r = filt_coeff.reshape(npairs, K)
    fcs = jnp.concatenate(
        [fc_r[order], jnp.zeros((npad - npairs, K), jnp.float32)])
    fcs = jnp.pad(fcs, ((0, 0), (0, 16 - K)))
    v0s = (sv[0::PB] // 8) * 8
    v0s = jnp.minimum(v0s, NROWS - RNG).astype(jnp.int32)
    return sv, pair_f, fcs, v0s


def _v2v_call(x, face, fc, prep, nf, W1, b1, W2, b2):
    fn = face.shape[0]
    xf = x[face]
    h = jnp.einsum('fvk,fvc->fkc', fc, xf).reshape(fn, -1)
    f1 = jax.nn.relu(h @ W1 + b1)
    c1 = f1.shape[1]
    sv, pair_f, fcs, v0s = prep
    f1g = f1[pair_f]
    nb = sv.shape[0] // PB
    acc = _scatter_acc(v0s, sv, fcs, f1g, c1)
    denom = jnp.maximum(nf, 1).astype(x.dtype)[:, None]
    v = acc[:x.shape[0]] / denom
    return jax.nn.relu(v @ W2 + b2)


def _pcloud(x, nn_cnt, nn_idx, coeff, W, b):
    xn = x[nn_idx]
    h = jnp.einsum('nmk,nmc->nkc', coeff, xn).reshape(x.shape[0], -1)
    h = h / jnp.maximum(nn_cnt, 1).astype(x.dtype)[:, None]
    return jax.nn.relu(h @ W + b)


def _final_kernel(x_ref, w_ref, b_ref, o_ref):
    o_ref[...] = jax.nn.relu(
        jnp.dot(x_ref[...], w_ref[...], preferred_element_type=jnp.float32)
        + b_ref[...])


def _final_dense(x, W, b):
    n, c = x.shape
    cout = W.shape[1]
    blk = 1024
    npad = ((n + blk - 1) // blk) * blk
    xp = jnp.pad(x, ((0, npad - n), (0, 0)))
    out = pl.pallas_call(
        _final_kernel,
        grid=(npad // blk,),
        in_specs=[
            pl.BlockSpec((blk, c), lambda i: (i, 0)),
            pl.BlockSpec((c, cout), lambda i: (0, 0)),
            pl.BlockSpec((cout,), lambda i: (0,)),
        ],
        out_specs=pl.BlockSpec((blk, cout), lambda i: (i, 0)),
        out_shape=jax.ShapeDtypeStruct((npad, cout), jnp.float32),
    )(xp, W, b)
    return out[:n]


def kernel(inputs, vertex, face, full_nf_count, full_vt_map, filt_coeff, nv_in, params):
    nn_cnt, nn_idx, coeff = _build_graph(vertex[:, :3], RADIUS, MAXNN)
    prep = _prep_pairs(face, filt_coeff)
    x = inputs
    for n in range(MAXITER):
        M = _v2v_call(x, face, filt_coeff, prep, full_nf_count,
                      params['m1_W1_%d' % n], params['m1_b1_%d' % n],
                      params['m1_W2_%d' % n], params['m1_b2_%d' % n])
        M = _v2v_call(M, face, filt_coeff, prep, full_nf_count,
                      params['m2_W1_%d' % n], params['m2_b1_%d' % n],
                      params['m2_W2_%d' % n], params['m2_b2_%d' % n])
        P = jax.nn.relu(x @ params['d_W_%d' % n] + params['d_b_%d' % n])
        P = _pcloud(P, nn_cnt, nn_idx, coeff, params['p_W_%d' % n], params['p_b_%d' % n])
        x = jnp.concatenate([x, M, P], axis=-1)
    return _final_dense(x, params['t_W'], params['t_b'])


# speedup vs baseline: 2.6243x; 2.6243x over previous
import functools

import jax
import jax.numpy as jnp
from jax.experimental import pallas as pl
from jax.experimental.pallas import tpu as pltpu

N = 10000
L = 2
K = (L + 1) ** 2
KP = K + 1
MAXNN = 32
RADIUS = 0.1
MAXITER = 2

NPAD = 10240
PB = 512
RNG = 1024
NROWS = 12288


def _sh2(u):
    x = u[..., 0]; y = u[..., 1]; z = u[..., 2]
    return jnp.stack([
        0.282095 * jnp.ones_like(x),
        0.488603 * y,
        0.488603 * z,
        0.488603 * x,
        1.092548 * x * y,
        1.092548 * y * z,
        0.315392 * (3.0 * z * z - 1.0),
        1.092548 * x * z,
        0.546274 * (x * x - y * y),
    ], axis=-1)


GB_ROWS = 128
NBUCKET = 131071
IMAX = 2**31 - 1


def _graph_select_body(q_ref, xt_ref, xn_ref, idx_ref, *, max_nn, scale, r2):
    i = pl.program_id(0)
    q = q_ref[...]
    qn = jnp.sum(q * q, axis=1, keepdims=True)
    ncol = xt_ref.shape[1]
    d2 = qn + xn_ref[...] - 2.0 * jnp.dot(q, xt_ref[...],
                                          preferred_element_type=jnp.float32)
    d2 = jnp.maximum(d2, 0.0)
    col = jax.lax.broadcasted_iota(jnp.int32, (GB_ROWS, ncol), 1)
    gid = i * GB_ROWS + jax.lax.broadcasted_iota(jnp.int32, (GB_ROWS, ncol), 0)
    bucket = jnp.where(d2 < r2,
                       jnp.minimum((d2 * scale).astype(jnp.int32),
                                   NBUCKET - 1),
                       NBUCKET)
    key = jnp.bitwise_or(jnp.left_shift(bucket, 14), col)
    key = jnp.where(col == gid, IMAX, key)
    cols = []
    for _ in range(max_nn):
        m = jnp.min(key, axis=1, keepdims=True)
        cols.append(jnp.bitwise_and(m, 0x3FFF))
        key = jnp.where(key == m, IMAX, key)
    idx_ref[...] = jnp.concatenate(cols, axis=1)


def _graph_select(xyz, radius, max_nn):
    n = xyz.shape[0]
    nrow = ((n + GB_ROWS - 1) // GB_ROWS) * GB_ROWS
    ncol = ((n + 127) // 128) * 128
    qc = jnp.pad(xyz, ((0, nrow - n), (0, 8 - 3)),
                 constant_values=0.0)
    xc = jnp.pad(xyz, ((0, ncol - n), (0, 8 - 3)), constant_values=0.0)
    xc = xc.at[n:, 0].set(1e3)
    xt = xc.T
    xn = jnp.sum(xc * xc, axis=1)[None, :]
    r2 = radius * radius
    scale = (NBUCKET - 1) / r2
    body = functools.partial(_graph_select_body, max_nn=max_nn,
                             scale=scale, r2=r2)
    nn_idx = pl.pallas_call(
        body,
        grid=(nrow // GB_ROWS,),
        in_specs=[
            pl.BlockSpec((GB_ROWS, 8), lambda i: (i, 0)),
            pl.BlockSpec((8, ncol), lambda i: (0, 0)),
            pl.BlockSpec((1, ncol), lambda i: (0, 0)),
        ],
        out_specs=pl.BlockSpec((GB_ROWS, max_nn), lambda i: (i, 0)),
        out_shape=jax.ShapeDtypeStruct((nrow, max_nn), jnp.int32),
    )(qc, xt, xn)
    return nn_idx[:n]


def _build_graph(xyz, radius, max_nn):
    nn_idx = _graph_select(xyz, radius, max_nn)
    dirs = xyz[nn_idx] - xyz[:, None, :]
    d2 = jnp.sum(dirs * dirs, -1)
    valid = d2 < radius * radius
    nn_cnt = valid.sum(-1)
    r = jnp.sqrt(jnp.maximum(d2, 1e-12))
    u = dirs / r[..., None]
    sh = _sh2(u)
    coeff = jnp.concatenate([sh, jnp.ones(sh.shape[:-1] + (1,), dtype=sh.dtype)], axis=-1)
    coeff = coeff * valid[..., None].astype(coeff.dtype)
    return nn_cnt, nn_idx, coeff


def _scatter_body(v0s_ref, sv_ref, fcs_ref, f1g_ref, acc_ref):
    b = pl.program_id(0)

    @pl.when(b == 0)
    def _():
        acc_ref[...] = jnp.zeros_like(acc_ref)

    c1 = f1g_ref.shape[1]
    v0 = pl.multiple_of(v0s_ref[b], 8)
    rel = sv_ref[0, :, :] - v0
    rel = jnp.clip(rel, 0, RNG - 1)
    onehot_t = (jax.lax.broadcasted_iota(jnp.int32, (RNG, PB), 0)
                == rel).astype(jnp.float32)
    f1g = f1g_ref[...]
    z = jnp.concatenate(
        [fcs_ref[:, k][:, None] * f1g for k in range(K)], axis=1)
    delta = jnp.dot(onehot_t, z, preferred_element_type=jnp.float32)
    cur = acc_ref[pl.ds(v0, RNG), :]
    acc_ref[pl.ds(v0, RNG), :] = cur + delta


@functools.partial(jax.jit, static_argnames=("c1",))
def _scatter_acc(v0s, sv3, fcs, f1g, c1):
    nb = sv3.shape[0] // PB
    grid_spec = pltpu.PrefetchScalarGridSpec(
        num_scalar_prefetch=1,
        grid=(nb,),
        in_specs=[
            pl.BlockSpec((1, 1, PB), lambda b, v0s: (b, 0, 0)),
            pl.BlockSpec((PB, 16), lambda b, v0s: (b, 0)),
            pl.BlockSpec((PB, c1), lambda b, v0s: (b, 0)),
        ],
        out_specs=pl.BlockSpec((NROWS, K * c1), lambda b, v0s: (0, 0)),
    )
    return pl.pallas_call(
        _scatter_body,
        grid_spec=grid_spec,
        out_shape=jax.ShapeDtypeStruct((NROWS, K * c1), jnp.float32),
        compiler_params=pltpu.CompilerParams(
            dimension_semantics=("arbitrary",)),
    )(v0s, sv3.reshape(nb, 1, PB), fcs, f1g)


def _prep_pairs(face, filt_coeff):
    f = face.shape[0]
    npairs = f * 3
    npad = ((npairs + PB - 1) // PB) * PB
    pair_v = face.reshape(-1)
    order = jnp.argsort(pair_v)
    sv = jnp.concatenate(
        [pair_v[order], jnp.full((npad - npairs,), NPAD, jnp.int32)])
    pair_f = jnp.concatenate(
        [(order // 3).astype(jnp.int32), jnp.zeros((npad - npairs,), jnp.int32)])
    fc_r = filt_coeff.reshape(npairs, K)
    fcs = jnp.concatenate(
        [fc_r[order], jnp.zeros((npad - npairs, K), jnp.float32)])
    fcs = jnp.pad(fcs, ((0, 0), (0, 16 - K)))
    v0s = (sv[0::PB] // 8) * 8
    v0s = jnp.minimum(v0s, NROWS - RNG).astype(jnp.int32)
    return sv, pair_f, fcs, v0s


def _v2v_call(x, face, fc, prep, nf, W1, b1, W2, b2):
    fn = face.shape[0]
    xf = x[face]
    h = jnp.einsum('fvk,fvc->fkc', fc, xf).reshape(fn, -1)
    f1 = jax.nn.relu(h @ W1 + b1)
    c1 = f1.shape[1]
    sv, pair_f, fcs, v0s = prep
    f1g = f1[pair_f]
    nb = sv.shape[0] // PB
    acc = _scatter_acc(v0s, sv, fcs, f1g, c1)
    denom = jnp.maximum(nf, 1).astype(x.dtype)[:, None]
    v = acc[:x.shape[0]] / denom
    return jax.nn.relu(v @ W2 + b2)


def _pcloud(x, nn_cnt, nn_idx, coeff, W, b):
    xn = x[nn_idx]
    h = jnp.einsum('nmk,nmc->nkc', coeff, xn).reshape(x.shape[0], -1)
    h = h / jnp.maximum(nn_cnt, 1).astype(x.dtype)[:, None]
    return jax.nn.relu(h @ W + b)


def _final_kernel(x_ref, w_ref, b_ref, o_ref):
    o_ref[...] = jax.nn.relu(
        jnp.dot(x_ref[...], w_ref[...], preferred_element_type=jnp.float32)
        + b_ref[...])


def _final_dense(x, W, b):
    n, c = x.shape
    cout = W.shape[1]
    blk = 1024
    npad = ((n + blk - 1) // blk) * blk
    xp = jnp.pad(x, ((0, npad - n), (0, 0)))
    out = pl.pallas_call(
        _final_kernel,
        grid=(npad // blk,),
        in_specs=[
            pl.BlockSpec((blk, c), lambda i: (i, 0)),
            pl.BlockSpec((c, cout), lambda i: (0, 0)),
            pl.BlockSpec((cout,), lambda i: (0,)),
        ],
        out_specs=pl.BlockSpec((blk, cout), lambda i: (i, 0)),
        out_shape=jax.ShapeDtypeStruct((npad, cout), jnp.float32),
    )(xp, W, b)
    return out[:n]


def kernel(inputs, vertex, face, full_nf_count, full_vt_map, filt_coeff, nv_in, params):
    nn_cnt, nn_idx, coeff = _build_graph(vertex[:, :3], RADIUS, MAXNN)
    prep = _prep_pairs(face, filt_coeff)
    x = inputs
    for n in range(MAXITER):
        M = _v2v_call(x, face, filt_coeff, prep, full_nf_count,
                      params['m1_W1_%d' % n], params['m1_b1_%d' % n],
                      params['m1_W2_%d' % n], params['m1_b2_%d' % n])
        M = _v2v_call(M, face, filt_coeff, prep, full_nf_count,
                      params['m2_W1_%d' % n], params['m2_b1_%d' % n],
                      params['m2_W2_%d' % n], params['m2_b2_%d' % n])
        P = jax.nn.relu(x @ params['d_W_%d' % n] + params['d_b_%d' % n])
        P = _pcloud(P, nn_cnt, nn_idx, coeff, params['p_W_%d' % n], params['p_b_%d' % n])
        x = jnp.concatenate([x, M, P], axis=-1)
    return _final_dense(x, params['t_W'], params['t_b'])


# speedup vs baseline: 3.2799x; 1.2498x over previous
import functools

import jax
import jax.numpy as jnp
from jax import lax
from jax.experimental import pallas as pl
from jax.experimental.pallas import tpu as pltpu
from jax.experimental.pallas import tpu_sc as plsc

N = 10000
L = 2
K = (L + 1) ** 2
KP = K + 1
MAXNN = 32
RADIUS = 0.1
MAXITER = 2

NPAD = 10240
PB = 512
RNG = 1024
NROWS = 12288


def _sh2(u):
    x = u[..., 0]; y = u[..., 1]; z = u[..., 2]
    return jnp.stack([
        0.282095 * jnp.ones_like(x),
        0.488603 * y,
        0.488603 * z,
        0.488603 * x,
        1.092548 * x * y,
        1.092548 * y * z,
        0.315392 * (3.0 * z * z - 1.0),
        1.092548 * x * z,
        0.546274 * (x * x - y * y),
    ], axis=-1)


NW = 32


def _sc_gather(table, idx, KCH):
    V, D = table.shape
    B = idx.shape[0]
    assert B % (NW * 128) == 0 and D % 16 == 0
    b_per_w = B // NW
    nir = b_per_w // 128
    assert nir % KCH == 0
    G = nir // KCH
    mesh = plsc.VectorSubcoreMesh(core_axis_name="c", subcore_axis_name="s")

    @functools.partial(
        pl.kernel, mesh=mesh,
        out_type=jax.ShapeDtypeStruct((B, D), jnp.float32),
        scratch_types=[
            pltpu.VMEM((KCH * 128,), jnp.int32),
            pltpu.VMEM((KCH * 128, D), jnp.float32),
            pltpu.SemaphoreType.DMA,
        ],
    )
    def k(table_hbm, idx_hbm, out_hbm, idx_v, rows_v, sem):
        wid = lax.axis_index("s") * 2 + lax.axis_index("c")

        def body(g, carry):
            row0 = wid * b_per_w + g * (KCH * 128)
            pltpu.sync_copy(idx_hbm.at[pl.ds(row0, KCH * 128)], idx_v)
            cps = [
                pltpu.async_copy(table_hbm.at[idx_v.at[pl.ds(j * 128, 128)]],
                                 rows_v.at[pl.ds(j * 128, 128)], sem)
                for j in range(KCH)
            ]
            for c in cps:
                c.wait()
            pltpu.sync_copy(rows_v, out_hbm.at[pl.ds(row0, KCH * 128)])
            return carry

        lax.fori_loop(0, G, body, 0)

    return k(table, idx)


def _gather_rows(table, idx):
    v, d = table.shape
    dpad = ((d + 127) // 128) * 128
    kch = 5 if dpad <= 128 else 3
    tp = table if dpad == d else jnp.pad(table, ((0, 0), (0, dpad - d)))
    b = idx.shape[0]
    mult = NW * 128 * kch
    bpad = ((b + mult - 1) // mult) * mult
    idxp = idx if bpad == b else jnp.pad(idx, (0, bpad - b))
    out = _sc_gather(tp, idxp, kch)
    return out[:b, :d]


GB_ROWS = 128
NBUCKET = 131071
IMAX = 2**31 - 1


def _graph_select_body(q_ref, xt_ref, xn_ref, idx_ref, *, max_nn, scale, r2):
    i = pl.program_id(0)
    q = q_ref[...]
    qn = jnp.sum(q * q, axis=1, keepdims=True)
    ncol = xt_ref.shape[1]
    d2 = qn + xn_ref[...] - 2.0 * jnp.dot(q, xt_ref[...],
                                          preferred_element_type=jnp.float32)
    d2 = jnp.maximum(d2, 0.0)
    col = jax.lax.broadcasted_iota(jnp.int32, (GB_ROWS, ncol), 1)
    gid = i * GB_ROWS + jax.lax.broadcasted_iota(jnp.int32, (GB_ROWS, ncol), 0)
    bucket = jnp.where(d2 < r2,
                       jnp.minimum((d2 * scale).astype(jnp.int32),
                                   NBUCKET - 1),
                       NBUCKET)
    key = jnp.bitwise_or(jnp.left_shift(bucket, 14), col)
    key = jnp.where(col == gid, IMAX, key)
    cols = []
    for _ in range(max_nn):
        m = jnp.min(key, axis=1, keepdims=True)
        cols.append(jnp.bitwise_and(m, 0x3FFF))
        key = jnp.where(key == m, IMAX, key)
    idx_ref[...] = jnp.concatenate(cols, axis=1)


def _graph_select(xyz, radius, max_nn):
    n = xyz.shape[0]
    nrow = ((n + GB_ROWS - 1) // GB_ROWS) * GB_ROWS
    ncol = ((n + 127) // 128) * 128
    qc = jnp.pad(xyz, ((0, nrow - n), (0, 8 - 3)),
                 constant_values=0.0)
    xc = jnp.pad(xyz, ((0, ncol - n), (0, 8 - 3)), constant_values=0.0)
    xc = xc.at[n:, 0].set(1e3)
    xt = xc.T
    xn = jnp.sum(xc * xc, axis=1)[None, :]
    r2 = radius * radius
    scale = (NBUCKET - 1) / r2
    body = functools.partial(_graph_select_body, max_nn=max_nn,
                             scale=scale, r2=r2)
    nn_idx = pl.pallas_call(
        body,
        grid=(nrow // GB_ROWS,),
        in_specs=[
            pl.BlockSpec((GB_ROWS, 8), lambda i: (i, 0)),
            pl.BlockSpec((8, ncol), lambda i: (0, 0)),
            pl.BlockSpec((1, ncol), lambda i: (0, 0)),
        ],
        out_specs=pl.BlockSpec((GB_ROWS, max_nn), lambda i: (i, 0)),
        out_shape=jax.ShapeDtypeStruct((nrow, max_nn), jnp.int32),
    )(qc, xt, xn)
    return nn_idx[:n]


def _build_graph(xyz, radius, max_nn):
    nn_idx = _graph_select(xyz, radius, max_nn)
    dirs = xyz[nn_idx] - xyz[:, None, :]
    d2 = jnp.sum(dirs * dirs, -1)
    valid = d2 < radius * radius
    nn_cnt = valid.sum(-1)
    r = jnp.sqrt(jnp.maximum(d2, 1e-12))
    u = dirs / r[..., None]
    sh = _sh2(u)
    coeff = jnp.concatenate([sh, jnp.ones(sh.shape[:-1] + (1,), dtype=sh.dtype)], axis=-1)
    coeff = coeff * valid[..., None].astype(coeff.dtype)
    return nn_cnt, nn_idx, coeff


def _scatter_body(v0s_ref, sv_ref, fcs_ref, f1g_ref, acc_ref):
    b = pl.program_id(0)

    @pl.when(b == 0)
    def _():
        acc_ref[...] = jnp.zeros_like(acc_ref)

    c1 = f1g_ref.shape[1]
    v0 = pl.multiple_of(v0s_ref[b], 8)
    rel = sv_ref[0, :, :] - v0
    rel = jnp.clip(rel, 0, RNG - 1)
    onehot_t = (jax.lax.broadcasted_iota(jnp.int32, (RNG, PB), 0)
                == rel).astype(jnp.float32)
    f1g = f1g_ref[...]
    z = jnp.concatenate(
        [fcs_ref[:, k][:, None] * f1g for k in range(K)], axis=1)
    delta = jnp.dot(onehot_t, z, preferred_element_type=jnp.float32)
    cur = acc_ref[pl.ds(v0, RNG), :]
    acc_ref[pl.ds(v0, RNG), :] = cur + delta


@functools.partial(jax.jit, static_argnames=("c1",))
def _scatter_acc(v0s, sv3, fcs, f1g, c1):
    nb = sv3.shape[0] // PB
    grid_spec = pltpu.PrefetchScalarGridSpec(
        num_scalar_prefetch=1,
        grid=(nb,),
        in_specs=[
            pl.BlockSpec((1, 1, PB), lambda b, v0s: (b, 0, 0)),
            pl.BlockSpec((PB, 16), lambda b, v0s: (b, 0)),
            pl.BlockSpec((PB, c1), lambda b, v0s: (b, 0)),
        ],
        out_specs=pl.BlockSpec((NROWS, K * c1), lambda b, v0s: (0, 0)),
    )
    return pl.pallas_call(
        _scatter_body,
        grid_spec=grid_spec,
        out_shape=jax.ShapeDtypeStruct((NROWS, K * c1), jnp.float32),
        compiler_params=pltpu.CompilerParams(
            dimension_semantics=("arbitrary",)),
    )(v0s, sv3.reshape(nb, 1, PB), fcs, f1g)


def _prep_pairs(face, filt_coeff):
    f = face.shape[0]
    npairs = f * 3
    npad = ((npairs + NW * 128 - 1) // (NW * 128)) * (NW * 128)
    assert npad % PB == 0
    pair_v = face.reshape(-1)
    order = jnp.argsort(pair_v)
    sv = jnp.concatenate(
        [pair_v[order], jnp.full((npad - npairs,), NPAD, jnp.int32)])
    pair_f = jnp.concatenate(
        [(order // 3).astype(jnp.int32), jnp.zeros((npad - npairs,), jnp.int32)])
    fc_r = filt_coeff.reshape(npairs, K)
    fcs = jnp.concatenate(
        [fc_r[order], jnp.zeros((npad - npairs, K), jnp.float32)])
    fcs = jnp.pad(fcs, ((0, 0), (0, 16 - K)))
    v0s = (sv[0::PB] // 8) * 8
    v0s = jnp.minimum(v0s, NROWS - RNG).astype(jnp.int32)
    face_flat = jnp.concatenate(
        [face.reshape(-1), jnp.zeros((npad - npairs,), jnp.int32)])
    fpad = npad // 3
    fc_pad = jnp.pad(filt_coeff.reshape(f, 3 * K),
                     ((0, fpad - f), (0, 32 - 3 * K)))
    return sv, pair_f, fcs, v0s, face_flat, fc_pad


def _v2v_call(x, face, fc, prep, nf, W1, b1, W2, b2):
    fn = face.shape[0]
    sv, pair_f, fcs, v0s, face_flat, fc_pad = prep
    npad = sv.shape[0]
    c = x.shape[1]
    xf3 = _gather_rows(x, face_flat).reshape(npad // 3, 3, c)
    f1 = _face_mlp(xf3, fc_pad, W1, b1)
    c1 = f1.shape[1]
    f1g = _gather_rows(f1, pair_f)
    acc = _scatter_acc(v0s, sv, fcs, f1g, c1)
    denom = jnp.maximum(nf, 1).astype(x.dtype)[:, None]
    v = acc[:x.shape[0]] / denom
    return jax.nn.relu(v @ W2 + b2)


def _pcloud(x, nn_cnt, nn_idx, coeff, W, b):
    n, c = x.shape
    npad = NPAD
    nn_p = jnp.pad(nn_idx, ((0, npad - n), (0, 0)))
    idx_mm = nn_p.T.reshape(-1)
    pg3 = _gather_rows(x, idx_mm).reshape(MAXNN, npad, c)
    cf_k = jnp.pad(coeff.transpose(2, 0, 1), ((0, 0), (0, npad - n), (0, 0)))
    cnt = jnp.pad(jnp.maximum(nn_cnt, 1).astype(jnp.float32),
                  (0, npad - n), constant_values=1.0)[:, None]
    return _pcloud_reduce(pg3, cf_k, cnt, W.reshape(KP, c, -1), b)[:n]


FB = 512


def _face_mlp_body(xf_ref, fc_ref, w1_ref, b1_ref, o_ref):
    xs = [xf_ref[:, s, :] for s in range(3)]
    hks = []
    for k in range(K):
        hk = xs[0] * fc_ref[:, k][:, None]
        hk += xs[1] * fc_ref[:, 9 + k][:, None]
        hk += xs[2] * fc_ref[:, 18 + k][:, None]
        hks.append(hk)
    h = jnp.concatenate(hks, axis=1)
    o_ref[...] = jax.nn.relu(
        jnp.dot(h, w1_ref[...], preferred_element_type=jnp.float32)
        + b1_ref[...])


def _face_mlp(xf3, fc_pad, W1, b1):
    fpad, _, c = xf3.shape
    c1 = W1.shape[1]
    return pl.pallas_call(
        _face_mlp_body,
        grid=(fpad // FB,),
        in_specs=[
            pl.BlockSpec((FB, 3, c), lambda i: (i, 0, 0)),
            pl.BlockSpec((FB, 32), lambda i: (i, 0)),
            pl.BlockSpec((K * c, c1), lambda i: (0, 0)),
            pl.BlockSpec((c1,), lambda i: (0,)),
        ],
        out_specs=pl.BlockSpec((FB, c1), lambda i: (i, 0)),
        out_shape=jax.ShapeDtypeStruct((fpad, c1), jnp.float32),
    )(xf3, fc_pad, W1, b1)


QB = 256


def _pcloud_body(pg_ref, cf_ref, cnt_ref, w_ref, b_ref, o_ref):
    k = pl.program_id(1)
    c = pg_ref.shape[2]
    cf = cf_ref[0]
    acc = jnp.zeros((QB, c), jnp.float32)
    for m in range(MAXNN):
        acc += pg_ref[m] * cf[:, m][:, None]
    part = jnp.dot(acc / cnt_ref[...], w_ref[0],
                   preferred_element_type=jnp.float32)

    @pl.when(k == 0)
    def _():
        o_ref[...] = part

    @pl.when(k > 0)
    def _():
        o_ref[...] += part

    @pl.when(k == KP - 1)
    def _():
        o_ref[...] = jax.nn.relu(o_ref[...] + b_ref[...])


def _pcloud_reduce(pg3, cf_k, cnt, pW3, pb):
    _, npad, c = pg3.shape
    c2 = pW3.shape[2]
    return pl.pallas_call(
        _pcloud_body,
        grid=(npad // QB, KP),
        in_specs=[
            pl.BlockSpec((MAXNN, QB, c), lambda i, k: (0, i, 0)),
            pl.BlockSpec((1, QB, MAXNN), lambda i, k: (k, i, 0)),
            pl.BlockSpec((QB, 1), lambda i, k: (i, 0)),
            pl.BlockSpec((1, c, c2), lambda i, k: (k, 0, 0)),
            pl.BlockSpec((c2,), lambda i, k: (0,)),
        ],
        out_specs=pl.BlockSpec((QB, c2), lambda i, k: (i, 0)),
        out_shape=jax.ShapeDtypeStruct((npad, c2), jnp.float32),
        compiler_params=pltpu.CompilerParams(
            dimension_semantics=("arbitrary", "arbitrary")),
    )(pg3, cf_k, cnt, pW3, pb)


def _final_kernel(x_ref, w_ref, b_ref, o_ref):
    o_ref[...] = jax.nn.relu(
        jnp.dot(x_ref[...], w_ref[...], preferred_element_type=jnp.float32)
        + b_ref[...])


def _final_dense(x, W, b):
    n, c = x.shape
    cout = W.shape[1]
    blk = 1024
    npad = ((n + blk - 1) // blk) * blk
    xp = jnp.pad(x, ((0, npad - n), (0, 0)))
    out = pl.pallas_call(
        _final_kernel,
        grid=(npad // blk,),
        in_specs=[
            pl.BlockSpec((blk, c), lambda i: (i, 0)),
            pl.BlockSpec((c, cout), lambda i: (0, 0)),
            pl.BlockSpec((cout,), lambda i: (0,)),
        ],
        out_specs=pl.BlockSpec((blk, cout), lambda i: (i, 0)),
        out_shape=jax.ShapeDtypeStruct((npad, cout), jnp.float32),
    )(xp, W, b)
    return out[:n]


def kernel(inputs, vertex, face, full_nf_count, full_vt_map, filt_coeff, nv_in, params):
    nn_cnt, nn_idx, coeff = _build_graph(vertex[:, :3], RADIUS, MAXNN)
    prep = _prep_pairs(face, filt_coeff)
    x = inputs
    for n in range(MAXITER):
        M = _v2v_call(x, face, filt_coeff, prep, full_nf_count,
                      params['m1_W1_%d' % n], params['m1_b1_%d' % n],
                      params['m1_W2_%d' % n], params['m1_b2_%d' % n])
        M = _v2v_call(M, face, filt_coeff, prep, full_nf_count,
                      params['m2_W1_%d' % n], params['m2_b1_%d' % n],
                      params['m2_W2_%d' % n], params['m2_b2_%d' % n])
        P = _final_dense(x, params['d_W_%d' % n], params['d_b_%d' % n])
        P = _pcloud(P, nn_cnt, nn_idx, coeff, params['p_W_%d' % n], params['p_b_%d' % n])
        x = jnp.concatenate([x, M, P], axis=-1)
    return _final_dense(x, params['t_W'], params['t_b'])


# speedup vs baseline: 3.3327x; 1.0161x over previous
import functools

import jax
import jax.numpy as jnp
from jax import lax
from jax.experimental import pallas as pl
from jax.experimental.pallas import tpu as pltpu
from jax.experimental.pallas import tpu_sc as plsc

N = 10000
L = 2
K = (L + 1) ** 2
KP = K + 1
MAXNN = 32
RADIUS = 0.1
MAXITER = 2

NPAD = 10240
PB = 512
RNG = 1024
NROWS = 12288


def _sh2(u):
    x = u[..., 0]; y = u[..., 1]; z = u[..., 2]
    return jnp.stack([
        0.282095 * jnp.ones_like(x),
        0.488603 * y,
        0.488603 * z,
        0.488603 * x,
        1.092548 * x * y,
        1.092548 * y * z,
        0.315392 * (3.0 * z * z - 1.0),
        1.092548 * x * z,
        0.546274 * (x * x - y * y),
    ], axis=-1)


NW = 32


def _sc_gather(table, idx, KCH):
    V, D = table.shape
    B = idx.shape[0]
    assert B % (NW * 128) == 0 and D % 16 == 0
    b_per_w = B // NW
    nir = b_per_w // 128
    assert nir % KCH == 0
    G = nir // KCH
    mesh = plsc.VectorSubcoreMesh(core_axis_name="c", subcore_axis_name="s")

    @functools.partial(
        pl.kernel, mesh=mesh,
        compiler_params=pltpu.CompilerParams(use_tc_tiling_on_sc=False),
        out_type=jax.ShapeDtypeStruct((B, D), jnp.float32),
        scratch_types=[
            pltpu.VMEM((KCH * 128,), jnp.int32),
            pltpu.VMEM((KCH * 128, D), jnp.float32),
            pltpu.SemaphoreType.DMA,
        ],
    )
    def k(table_hbm, idx_hbm, out_hbm, idx_v, rows_v, sem):
        wid = lax.axis_index("s") * 2 + lax.axis_index("c")

        def body(g, carry):
            row0 = wid * b_per_w + g * (KCH * 128)
            pltpu.sync_copy(idx_hbm.at[pl.ds(row0, KCH * 128)], idx_v)
            cps = [
                pltpu.async_copy(table_hbm.at[idx_v.at[pl.ds(j * 128, 128)]],
                                 rows_v.at[pl.ds(j * 128, 128)], sem)
                for j in range(KCH)
            ]
            for c in cps:
                c.wait()
            pltpu.sync_copy(rows_v, out_hbm.at[pl.ds(row0, KCH * 128)])
            return carry

        lax.fori_loop(0, G, body, 0)

    return k(table, idx)


def _gather_rows(table, idx):
    v, d = table.shape
    dpad = ((d + 63) // 64) * 64
    b = idx.shape[0]
    if dpad <= 64 and b % (NW * 128 * 10) == 0:
        kch = 10
    elif dpad <= 128:
        kch = 5
    else:
        kch = 3
    tp = table if dpad == d else jnp.pad(table, ((0, 0), (0, dpad - d)))
    mult = NW * 128 * kch
    bpad = ((b + mult - 1) // mult) * mult
    idxp = idx if bpad == b else jnp.pad(idx, (0, bpad - b))
    out = _sc_gather(tp, idxp, kch)
    return out[:b, :d]


GB_ROWS = 128
NBUCKET = 131071
IMAX = 2**31 - 1


def _graph_select_body(q_ref, xt_ref, xn_ref, idx_ref, *, max_nn, scale, r2):
    i = pl.program_id(0)
    q = q_ref[...]
    qn = jnp.sum(q * q, axis=1, keepdims=True)
    ncol = xt_ref.shape[1]
    d2 = qn + xn_ref[...] - 2.0 * jnp.dot(q, xt_ref[...],
                                          preferred_element_type=jnp.float32)
    d2 = jnp.maximum(d2, 0.0)
    col = jax.lax.broadcasted_iota(jnp.int32, (GB_ROWS, ncol), 1)
    gid = i * GB_ROWS + jax.lax.broadcasted_iota(jnp.int32, (GB_ROWS, ncol), 0)
    bucket = jnp.where(d2 < r2,
                       jnp.minimum((d2 * scale).astype(jnp.int32),
                                   NBUCKET - 1),
                       NBUCKET)
    key = jnp.bitwise_or(jnp.left_shift(bucket, 14), col)
    key = jnp.where(col == gid, IMAX, key)
    cols = []
    for _ in range(max_nn):
        m = jnp.min(key, axis=1, keepdims=True)
        cols.append(jnp.bitwise_and(m, 0x3FFF))
        key = jnp.where(key == m, IMAX, key)
    idx_ref[...] = jnp.concatenate(cols, axis=1)


def _graph_select(xyz, radius, max_nn):
    n = xyz.shape[0]
    nrow = ((n + GB_ROWS - 1) // GB_ROWS) * GB_ROWS
    ncol = ((n + 127) // 128) * 128
    qc = jnp.pad(xyz, ((0, nrow - n), (0, 8 - 3)),
                 constant_values=0.0)
    xc = jnp.pad(xyz, ((0, ncol - n), (0, 8 - 3)), constant_values=0.0)
    xc = xc.at[n:, 0].set(1e3)
    xt = xc.T
    xn = jnp.sum(xc * xc, axis=1)[None, :]
    r2 = radius * radius
    scale = (NBUCKET - 1) / r2
    body = functools.partial(_graph_select_body, max_nn=max_nn,
                             scale=scale, r2=r2)
    nn_idx = pl.pallas_call(
        body,
        grid=(nrow // GB_ROWS,),
        in_specs=[
            pl.BlockSpec((GB_ROWS, 8), lambda i: (i, 0)),
            pl.BlockSpec((8, ncol), lambda i: (0, 0)),
            pl.BlockSpec((1, ncol), lambda i: (0, 0)),
        ],
        out_specs=pl.BlockSpec((GB_ROWS, max_nn), lambda i: (i, 0)),
        out_shape=jax.ShapeDtypeStruct((nrow, max_nn), jnp.int32),
    )(qc, xt, xn)
    return nn_idx[:n]


def _build_graph(xyz, radius, max_nn):
    nn_idx = _graph_select(xyz, radius, max_nn)
    dirs = xyz[nn_idx] - xyz[:, None, :]
    d2 = jnp.sum(dirs * dirs, -1)
    valid = d2 < radius * radius
    nn_cnt = valid.sum(-1)
    r = jnp.sqrt(jnp.maximum(d2, 1e-12))
    u = dirs / r[..., None]
    sh = _sh2(u)
    coeff = jnp.concatenate([sh, jnp.ones(sh.shape[:-1] + (1,), dtype=sh.dtype)], axis=-1)
    coeff = coeff * valid[..., None].astype(coeff.dtype)
    return nn_cnt, nn_idx, coeff


def _scatter_body(v0s_ref, sv_ref, fcs_ref, f1g_ref, acc_ref):
    b = pl.program_id(0)

    @pl.when(b == 0)
    def _():
        acc_ref[...] = jnp.zeros_like(acc_ref)

    c1 = f1g_ref.shape[1]
    v0 = pl.multiple_of(v0s_ref[b], 8)
    rel = sv_ref[0, :, :] - v0
    rel = jnp.clip(rel, 0, RNG - 1)
    onehot_t = (jax.lax.broadcasted_iota(jnp.int32, (RNG, PB), 0)
                == rel).astype(jnp.float32)
    f1g = f1g_ref[...]
    z = jnp.concatenate(
        [fcs_ref[:, k][:, None] * f1g for k in range(K)], axis=1)
    delta = jnp.dot(onehot_t, z, preferred_element_type=jnp.float32)
    cur = acc_ref[pl.ds(v0, RNG), :]
    acc_ref[pl.ds(v0, RNG), :] = cur + delta


@functools.partial(jax.jit, static_argnames=("c1",))
def _scatter_acc(v0s, sv3, fcs, f1g, c1):
    nb = sv3.shape[0] // PB
    grid_spec = pltpu.PrefetchScalarGridSpec(
        num_scalar_prefetch=1,
        grid=(nb,),
        in_specs=[
            pl.BlockSpec((1, 1, PB), lambda b, v0s: (b, 0, 0)),
            pl.BlockSpec((PB, 16), lambda b, v0s: (b, 0)),
            pl.BlockSpec((PB, c1), lambda b, v0s: (b, 0)),
        ],
        out_specs=pl.BlockSpec((NROWS, K * c1), lambda b, v0s: (0, 0)),
    )
    return pl.pallas_call(
        _scatter_body,
        grid_spec=grid_spec,
        out_shape=jax.ShapeDtypeStruct((NROWS, K * c1), jnp.float32),
        compiler_params=pltpu.CompilerParams(
            dimension_semantics=("arbitrary",)),
    )(v0s, sv3.reshape(nb, 1, PB), fcs, f1g)


def _prep_pairs(face, filt_coeff):
    f = face.shape[0]
    npairs = f * 3
    npad = ((npairs + NW * 128 - 1) // (NW * 128)) * (NW * 128)
    assert npad % PB == 0
    pair_v = face.reshape(-1)
    order = jnp.argsort(pair_v)
    sv = jnp.concatenate(
        [pair_v[order], jnp.full((npad - npairs,), NPAD, jnp.int32)])
    pair_f = jnp.concatenate(
        [(order // 3).astype(jnp.int32), jnp.zeros((npad - npairs,), jnp.int32)])
    fc_r = filt_coeff.reshape(npairs, K)
    fcs = jnp.concatenate(
        [fc_r[order], jnp.zeros((npad - npairs, K), jnp.float32)])
    fcs = jnp.pad(fcs, ((0, 0), (0, 16 - K)))
    v0s = (sv[0::PB] // 8) * 8
    v0s = jnp.minimum(v0s, NROWS - RNG).astype(jnp.int32)
    face_flat = jnp.concatenate(
        [face.reshape(-1), jnp.zeros((npad - npairs,), jnp.int32)])
    fpad = npad // 3
    fc_pad = jnp.pad(filt_coeff.reshape(f, 3 * K),
                     ((0, fpad - f), (0, 32 - 3 * K)))
    return sv, pair_f, fcs, v0s, face_flat, fc_pad


def _v2v_call(x, face, fc, prep, nf, W1, b1, W2, b2):
    fn = face.shape[0]
    sv, pair_f, fcs, v0s, face_flat, fc_pad = prep
    npad = sv.shape[0]
    c = x.shape[1]
    xf3 = _gather_rows(x, face_flat).reshape(npad // 3, 3, c)
    f1 = _face_mlp(xf3, fc_pad, W1, b1)
    c1 = f1.shape[1]
    f1g = _gather_rows(f1, pair_f)
    acc = _scatter_acc(v0s, sv, fcs, f1g, c1)
    denom = jnp.maximum(nf, 1).astype(x.dtype)[:, None]
    v = acc[:x.shape[0]] / denom
    return jax.nn.relu(v @ W2 + b2)


def _pcloud(x, nn_cnt, nn_idx, coeff, W, b):
    n, c = x.shape
    npad = NPAD
    nn_p = jnp.pad(nn_idx, ((0, npad - n), (0, 0)))
    idx_mm = nn_p.T.reshape(-1)
    pg3 = _gather_rows(x, idx_mm).reshape(MAXNN, npad, c)
    cf_k = jnp.pad(coeff.transpose(2, 0, 1), ((0, 0), (0, npad - n), (0, 0)))
    cnt = jnp.pad(jnp.maximum(nn_cnt, 1).astype(jnp.float32),
                  (0, npad - n), constant_values=1.0)[:, None]
    return _pcloud_reduce(pg3, cf_k, cnt, W.reshape(KP, c, -1), b)[:n]


FB = 512


def _face_mlp_body(xf_ref, fc_ref, w1_ref, b1_ref, o_ref):
    xs = [xf_ref[:, s, :] for s in range(3)]
    hks = []
    for k in range(K):
        hk = xs[0] * fc_ref[:, k][:, None]
        hk += xs[1] * fc_ref[:, 9 + k][:, None]
        hk += xs[2] * fc_ref[:, 18 + k][:, None]
        hks.append(hk)
    h = jnp.concatenate(hks, axis=1)
    o_ref[...] = jax.nn.relu(
        jnp.dot(h, w1_ref[...], preferred_element_type=jnp.float32)
        + b1_ref[...])


def _face_mlp(xf3, fc_pad, W1, b1):
    fpad, _, c = xf3.shape
    c1 = W1.shape[1]
    return pl.pallas_call(
        _face_mlp_body,
        grid=(fpad // FB,),
        in_specs=[
            pl.BlockSpec((FB, 3, c), lambda i: (i, 0, 0)),
            pl.BlockSpec((FB, 32), lambda i: (i, 0)),
            pl.BlockSpec((K * c, c1), lambda i: (0, 0)),
            pl.BlockSpec((c1,), lambda i: (0,)),
        ],
        out_specs=pl.BlockSpec((FB, c1), lambda i: (i, 0)),
        out_shape=jax.ShapeDtypeStruct((fpad, c1), jnp.float32),
    )(xf3, fc_pad, W1, b1)


QB = 256


def _pcloud_body(pg_ref, cf_ref, cnt_ref, w_ref, b_ref, o_ref):
    k = pl.program_id(1)
    c = pg_ref.shape[2]
    cf = cf_ref[0]
    acc = jnp.zeros((QB, c), jnp.float32)
    for m in range(MAXNN):
        acc += pg_ref[m] * cf[:, m][:, None]
    part = jnp.dot(acc / cnt_ref[...], w_ref[0],
                   preferred_element_type=jnp.float32)

    @pl.when(k == 0)
    def _():
        o_ref[...] = part

    @pl.when(k > 0)
    def _():
        o_ref[...] += part

    @pl.when(k == KP - 1)
    def _():
        o_ref[...] = jax.nn.relu(o_ref[...] + b_ref[...])


def _pcloud_reduce(pg3, cf_k, cnt, pW3, pb):
    _, npad, c = pg3.shape
    c2 = pW3.shape[2]
    return pl.pallas_call(
        _pcloud_body,
        grid=(npad // QB, KP),
        in_specs=[
            pl.BlockSpec((MAXNN, QB, c), lambda i, k: (0, i, 0)),
            pl.BlockSpec((1, QB, MAXNN), lambda i, k: (k, i, 0)),
            pl.BlockSpec((QB, 1), lambda i, k: (i, 0)),
            pl.BlockSpec((1, c, c2), lambda i, k: (k, 0, 0)),
            pl.BlockSpec((c2,), lambda i, k: (0,)),
        ],
        out_specs=pl.BlockSpec((QB, c2), lambda i, k: (i, 0)),
        out_shape=jax.ShapeDtypeStruct((npad, c2), jnp.float32),
        compiler_params=pltpu.CompilerParams(
            dimension_semantics=("arbitrary", "arbitrary")),
    )(pg3, cf_k, cnt, pW3, pb)


def _final_kernel(x_ref, w_ref, b_ref, o_ref):
    o_ref[...] = jax.nn.relu(
        jnp.dot(x_ref[...], w_ref[...], preferred_element_type=jnp.float32)
        + b_ref[...])


def _final_dense(x, W, b):
    n, c = x.shape
    cout = W.shape[1]
    blk = 1024
    npad = ((n + blk - 1) // blk) * blk
    xp = jnp.pad(x, ((0, npad - n), (0, 0)))
    out = pl.pallas_call(
        _final_kernel,
        grid=(npad // blk,),
        in_specs=[
            pl.BlockSpec((blk, c), lambda i: (i, 0)),
            pl.BlockSpec((c, cout), lambda i: (0, 0)),
            pl.BlockSpec((cout,), lambda i: (0,)),
        ],
        out_specs=pl.BlockSpec((blk, cout), lambda i: (i, 0)),
        out_shape=jax.ShapeDtypeStruct((npad, cout), jnp.float32),
    )(xp, W, b)
    return out[:n]


def kernel(inputs, vertex, face, full_nf_count, full_vt_map, filt_coeff, nv_in, params):
    nn_cnt, nn_idx, coeff = _build_graph(vertex[:, :3], RADIUS, MAXNN)
    prep = _prep_pairs(face, filt_coeff)
    x = inputs
    for n in range(MAXITER):
        M = _v2v_call(x, face, filt_coeff, prep, full_nf_count,
                      params['m1_W1_%d' % n], params['m1_b1_%d' % n],
                      params['m1_W2_%d' % n], params['m1_b2_%d' % n])
        M = _v2v_call(M, face, filt_coeff, prep, full_nf_count,
                      params['m2_W1_%d' % n], params['m2_b1_%d' % n],
                      params['m2_W2_%d' % n], params['m2_b2_%d' % n])
        P = _final_dense(x, params['d_W_%d' % n], params['d_b_%d' % n])
        P = _pcloud(P, nn_cnt, nn_idx, coeff, params['p_W_%d' % n], params['p_b_%d' % n])
        x = jnp.concatenate([x, M, P], axis=-1)
    return _final_dense(x, params['t_W'], params['t_b'])
